# baseline (speedup 1.0000x reference)
import jax
import jax.numpy as jnp
from jax.experimental import pallas as pl
from jax.experimental.pallas import tpu as pltpu

_MARGIN = 0.3
_BBLK = 1024


def _body(logits_ref, labels_ref, labels_t_ref, wf_ref, out_ref):
    c = pl.program_id(1)

    @pl.when(c == 0)
    def _init():
        logits = logits_ref[...]
        labels = labels_ref[...]
        bce = jnp.maximum(logits, 0.0) - logits * labels + jnp.log1p(
            jnp.exp(-jnp.abs(logits)))
        out_ref[...] = jnp.sum(bce, axis=1)

    wfb = wf_ref[0]
    max_sim = jnp.max(wfb, axis=1)
    rej = jnp.maximum(jax.nn.sigmoid(max_sim) - _MARGIN, 0.0)
    mask = (labels_t_ref[0, 0] == 0.0).astype(jnp.float32)
    out_ref[...] += rej * mask


def kernel(logits, wf, labels):
    B, C = logits.shape
    D = wf.shape[2]
    labels_t = labels.T.reshape(C, 1, B)
    grid = (B // _BBLK, C)
    return pl.pallas_call(
        _body,
        grid=grid,
        in_specs=[
            pl.BlockSpec((_BBLK, C), lambda i, c: (i, 0)),
            pl.BlockSpec((_BBLK, C), lambda i, c: (i, 0)),
            pl.BlockSpec((1, 1, _BBLK), lambda i, c: (c, 0, i)),
            pl.BlockSpec((1, _BBLK, D), lambda i, c: (c, i, 0)),
        ],
        out_specs=pl.BlockSpec((_BBLK,), lambda i, c: (i,)),
        out_shape=jax.ShapeDtypeStruct((B,), jnp.float32),
    )(logits, labels, labels_t, wf)


# speedup vs baseline: 1.5813x; 1.5813x over previous
import jax
import jax.numpy as jnp
from jax.experimental import pallas as pl
from jax.experimental.pallas import tpu as pltpu

_MARGIN = 0.3
_BBLK = 512
_CBLK = 8


def _rej_body(labels_t_ref, wf_ref, out_ref):
    j = pl.program_id(1)
    wfb = wf_ref[...]
    max_sim = jnp.max(wfb, axis=2)
    rej = jnp.maximum(jax.nn.sigmoid(max_sim) - _MARGIN, 0.0)
    mask = (labels_t_ref[...] == 0.0).astype(jnp.float32)
    part = jnp.sum(rej * mask, axis=0, keepdims=True)[None]

    @pl.when(j == 0)
    def _init():
        out_ref[...] = part

    @pl.when(j > 0)
    def _acc():
        out_ref[...] += part


def _bce_body(logits_ref, labels_ref, out_ref):
    logits = logits_ref[...]
    labels = labels_ref[...]
    bce = jnp.maximum(logits, 0.0) - logits * labels + jnp.log1p(
        jnp.exp(-jnp.abs(logits)))
    out_ref[...] = jnp.sum(bce, axis=1).reshape(1, 1, -1)


def kernel(logits, wf, labels):
    B, C = logits.shape
    D = wf.shape[2]
    labels_t = labels.T.reshape(C, B)

    rej = pl.pallas_call(
        _rej_body,
        grid=(B // _BBLK, C // _CBLK),
        in_specs=[
            pl.BlockSpec((_CBLK, _BBLK), lambda i, j: (j, i)),
            pl.BlockSpec((_CBLK, _BBLK, D), lambda i, j: (j, i, 0)),
        ],
        out_specs=pl.BlockSpec((1, 1, _BBLK), lambda i, j: (i, 0, 0)),
        out_shape=jax.ShapeDtypeStruct((B // _BBLK, 1, _BBLK), jnp.float32),
    )(labels_t, wf)

    bce = pl.pallas_call(
        _bce_body,
        grid=(B // _BBLK,),
        in_specs=[
            pl.BlockSpec((_BBLK, C), lambda i: (i, 0)),
            pl.BlockSpec((_BBLK, C), lambda i: (i, 0)),
        ],
        out_specs=pl.BlockSpec((1, 1, _BBLK), lambda i: (i, 0, 0)),
        out_shape=jax.ShapeDtypeStruct((B // _BBLK, 1, _BBLK), jnp.float32),
    )(logits, labels)

    return (rej + bce).reshape(B)
